# baseline (speedup 1.0000x reference)
import functools

import jax
import jax.numpy as jnp
from jax import lax
from jax.experimental import pallas as pl
from jax.experimental.pallas import tpu as pltpu
from jax.experimental.pallas import tpu_sc as plsc

N = 10000
NP = 10240
E = 320000
D = 128
H = 256
NC, NS, L = 2, 16, 16
RPW = 80
EROWS = NC * NS * RPW
RPS = EROWS // NS
_IDXC = 16
RPT = NP // NS

F32 = jnp.float32


@functools.lru_cache(maxsize=None)
def _sc_mesh():
  return plsc.VectorSubcoreMesh(
      core_axis_name="c", subcore_axis_name="s", num_cores=NC, num_subcores=NS)


def _zero_block(buf, nrows, ncols):
  def fill(i, _):
    for c in range(ncols // L):
      buf[i, c * L:(c + 1) * L] = jnp.zeros((L,), F32)
    return 0
  lax.fori_loop(0, nrows, fill, 0)


def _sc_degrees_body(src_hbm, dst_hbm, cnt_out_hbm, cnt_in_hbm,
                     idx_v, ones_v, zeros_v, accum):
  cid = lax.axis_index("c")
  sid = lax.axis_index("s")

  _zero_block(zeros_v, 128, 128)

  def fill(i, _):
    for c in range(128 // L):
      ones_v[i, c * L:(c + 1) * L] = jnp.ones((L,), F32)
    return 0
  lax.fori_loop(0, 128, fill, 0)

  base = sid * RPT
  for k in range(RPT // 128):
    pltpu.sync_copy(zeros_v, accum.at[pl.ds(base + k * 128, 128)])
  plsc.subcore_barrier()

  def run(e_hbm):
    def outer(oc, _):
      ebase = sid * RPS + oc * _IDXC
      pltpu.sync_copy(e_hbm.at[pl.ds(ebase, _IDXC)], idx_v)

      def step(j, _):
        pltpu.sync_copy(ones_v, accum.at[idx_v.at[j]], add=True)
        return 0
      lax.fori_loop(0, _IDXC, step, 0)
      return 0
    lax.fori_loop(0, RPS // _IDXC, outer, 0)

  @pl.when(cid == 0)
  def _():
    run(src_hbm)

  @pl.when(cid == 1)
  def _():
    run(dst_hbm)

  plsc.subcore_barrier()

  @pl.when(cid == 0)
  def _():
    pltpu.sync_copy(accum.at[pl.ds(base, RPT)], cnt_out_hbm.at[pl.ds(base, RPT)])

  @pl.when(cid == 1)
  def _():
    pltpu.sync_copy(accum.at[pl.ds(base, RPT)], cnt_in_hbm.at[pl.ds(base, RPT)])


def _sc_agg_body(h_l, h_r, src_hbm, dst_hbm, out_l, out_r,
                 idx_s, idx_d, rows_a, rows_b, gsem_a, gsem_b,
                 ssem_a, ssem_b, accum):
  cid = lax.axis_index("c")
  sid = lax.axis_index("s")

  _zero_block(rows_a, 128, 128)
  base = sid * RPT
  for k in range(RPT // 128):
    pltpu.sync_copy(rows_a, accum.at[pl.ds(base + k * 128, 128)])
  plsc.subcore_barrier()

  npairs = _IDXC // 2

  def run(h_ref):
    def outer(oc, _):
      ebase = sid * RPS + oc * _IDXC
      pltpu.sync_copy(src_hbm.at[pl.ds(ebase, _IDXC)], idx_s)
      pltpu.sync_copy(dst_hbm.at[pl.ds(ebase, _IDXC)], idx_d)

      def pair(p, _):
        j0 = 2 * p

        @pl.when(p == 0)
        def _():
          pltpu.async_copy(h_ref.at[idx_s.at[j0]], rows_a, gsem_a)
          pltpu.async_copy(h_ref.at[idx_s.at[j0 + 1]], rows_b, gsem_b)

        pltpu.make_async_copy(h_ref.at[idx_s.at[j0]], rows_a, gsem_a).wait()
        pltpu.async_copy(rows_a, accum.at[idx_d.at[j0]], ssem_a, add=True)
        pltpu.make_async_copy(h_ref.at[idx_s.at[j0 + 1]], rows_b, gsem_b).wait()
        pltpu.async_copy(rows_b, accum.at[idx_d.at[j0 + 1]], ssem_b, add=True)

        pltpu.make_async_copy(rows_a, accum.at[idx_d.at[j0]], ssem_a).wait()
        pltpu.make_async_copy(rows_b, accum.at[idx_d.at[j0 + 1]], ssem_b).wait()

        @pl.when(p < npairs - 1)
        def _():
          pltpu.async_copy(h_ref.at[idx_s.at[j0 + 2]], rows_a, gsem_a)
          pltpu.async_copy(h_ref.at[idx_s.at[j0 + 3]], rows_b, gsem_b)
        return 0
      lax.fori_loop(0, npairs, pair, 0)
      return 0
    lax.fori_loop(0, RPS // _IDXC, outer, 0)

  @pl.when(cid == 0)
  def _():
    run(h_l)

  @pl.when(cid == 1)
  def _():
    run(h_r)

  plsc.subcore_barrier()

  @pl.when(cid == 0)
  def _():
    pltpu.sync_copy(accum.at[pl.ds(base, RPT)], out_l.at[pl.ds(base, RPT)])

  @pl.when(cid == 1)
  def _():
    pltpu.sync_copy(accum.at[pl.ds(base, RPT)], out_r.at[pl.ds(base, RPT)])


def _sc_qgather_body(z_f, z_s, q_hbm, out_f, out_s, idx_v, rows_v, sem):
  cid = lax.axis_index("c")
  sid = lax.axis_index("s")
  w = sid * NC + cid
  pltpu.sync_copy(q_hbm.at[pl.ds((w // 2) * 8, 8)], idx_v)
  half = (w % 2) * 4

  def run(src_ref, dst_ref):
    def step(j, _):
      pltpu.async_copy(src_ref.at[idx_v.at[half + j]], rows_v, sem).wait()
      pltpu.sync_copy(rows_v, dst_ref.at[pl.ds(w * 512 + j * 128, 128)])
      return 0
    lax.fori_loop(0, 4, step, 0)

  run(z_f, out_f)
  run(z_s, out_s)


@functools.lru_cache(maxsize=None)
def _sc_built():
  mesh = _sc_mesh()
  degrees = pl.kernel(
      _sc_degrees_body,
      out_type=(jax.ShapeDtypeStruct((NP, 128), F32),
                jax.ShapeDtypeStruct((NP, 128), F32)),
      mesh=mesh,
      scratch_types=[
          pltpu.VMEM((_IDXC, 128), jnp.int32),
          pltpu.VMEM((128, 128), F32),
          pltpu.VMEM((128, 128), F32),
          pltpu.VMEM_SHARED((NP, 128), F32),
      ],
  )
  agg = pl.kernel(
      _sc_agg_body,
      out_type=(jax.ShapeDtypeStruct((NP, 128), F32),
                jax.ShapeDtypeStruct((NP, 128), F32)),
      mesh=mesh,
      scratch_types=[
          pltpu.VMEM((_IDXC, 128), jnp.int32),
          pltpu.VMEM((_IDXC, 128), jnp.int32),
          pltpu.VMEM((128, 128), F32),
          pltpu.VMEM((128, 128), F32),
          pltpu.SemaphoreType.DMA,
          pltpu.SemaphoreType.DMA,
          pltpu.SemaphoreType.DMA,
          pltpu.SemaphoreType.DMA,
          pltpu.VMEM_SHARED((NP, 128), F32),
      ],
  )
  qgather = pl.kernel(
      _sc_qgather_body,
      out_type=(jax.ShapeDtypeStruct((16384, H), F32),
                jax.ShapeDtypeStruct((16384, H), F32)),
      mesh=mesh,
      scratch_types=[
          pltpu.VMEM((8, 128), jnp.int32),
          pltpu.VMEM((128, H), F32),
          pltpu.SemaphoreType.DMA,
      ],
  )
  return degrees, agg, qgather


def _sc_degrees(*args):
  return _sc_built()[0](*args)


def _sc_agg(*args):
  return _sc_built()[1](*args)


def _sc_qgather(*args):
  return _sc_built()[2](*args)


def _deg_scale(c):
  return lax.rsqrt(jnp.maximum(c[:, 0:1], 1.0))


def _row_mask(i, rows):
  rid = i * rows + lax.broadcasted_iota(jnp.int32, (rows, 1), 0)
  return rid < N


def _elu(x):
  return jnp.where(x > 0, x, jnp.exp(x) - 1.0)


def _tc_prep_body(xf, xs, co, hf, hs):
  i = pl.program_id(0)
  so = _deg_scale(co[...])
  m = _row_mask(i, 1024)
  hf[...] = jnp.where(m, xf[...] * so, 0.0)
  hs[...] = jnp.where(m, xs[...] * so, 0.0)


def _tc_l1_body(al, ar, ci, co, w1f, b1f, w1s, b1s,
                ofl, ofr, osl, osr):
  i = pl.program_id(0)
  di = _deg_scale(ci[...])
  do = _deg_scale(co[...])
  m = _row_mask(i, 1024)
  hf = _elu(jnp.dot(al[...] * di, w1f[...],
                    preferred_element_type=F32) + b1f[...])
  hf = jnp.where(m, hf * do, 0.0)
  ofl[...] = hf[:, :128]
  ofr[...] = hf[:, 128:]
  hs = _elu(jnp.dot(ar[...] * di, w1s[...],
                    preferred_element_type=F32) + b1s[...])
  hs = jnp.where(m, hs * do, 0.0)
  osl[...] = hs[:, :128]
  osr[...] = hs[:, 128:]


def _tc_l2_body(fl, fr, sl, sr, ci, w2f, b2f, w2s, b2s, zf, zs):
  di = _deg_scale(ci[...])
  wf = w2f[...]
  ws = w2s[...]
  zf[...] = (jnp.dot(fl[...] * di, wf[:128, :], preferred_element_type=F32)
             + jnp.dot(fr[...] * di, wf[128:, :], preferred_element_type=F32)
             + b2f[...])
  zs[...] = (jnp.dot(sl[...] * di, ws[:128, :], preferred_element_type=F32)
             + jnp.dot(sr[...] * di, ws[128:, :], preferred_element_type=F32)
             + b2s[...])


def _tc_mlp_body(zqf, zqs, wd1_sf, bd1_sf, wd2_sf, bd2_sf,
                 wd1_fs, bd1_fs, wd2_fs, bd2_fs, zsf, zfs):
  h = _elu(jnp.dot(zqs[...], wd1_sf[...], preferred_element_type=F32)
           + bd1_sf[...])
  zsf[...] = _elu(jnp.dot(h, wd2_sf[...], preferred_element_type=F32)
                  + bd2_sf[...])
  h = _elu(jnp.dot(zqf[...], wd1_fs[...], preferred_element_type=F32)
           + bd1_fs[...])
  zfs[...] = _elu(jnp.dot(h, wd2_fs[...], preferred_element_type=F32)
                  + bd2_fs[...])


def _rowblock(rows, cols):
  return pl.BlockSpec((rows, cols), lambda i: (i, 0))


def _full(shape):
  return pl.BlockSpec(shape, lambda i: tuple(0 for _ in shape))


_tc_prep = pl.pallas_call(
    _tc_prep_body,
    grid=(NP // 1024,),
    in_specs=[_rowblock(1024, D), _rowblock(1024, D), _rowblock(1024, 128)],
    out_specs=[_rowblock(1024, D), _rowblock(1024, D)],
    out_shape=(jax.ShapeDtypeStruct((NP, D), F32),
               jax.ShapeDtypeStruct((NP, D), F32)),
)

_tc_l1 = pl.pallas_call(
    _tc_l1_body,
    grid=(NP // 1024,),
    in_specs=[_rowblock(1024, 128), _rowblock(1024, 128),
              _rowblock(1024, 128), _rowblock(1024, 128),
              _full((D, H)), _full((1, H)), _full((D, H)), _full((1, H))],
    out_specs=[_rowblock(1024, 128)] * 4,
    out_shape=tuple(jax.ShapeDtypeStruct((NP, 128), F32) for _ in range(4)),
)

_tc_l2 = pl.pallas_call(
    _tc_l2_body,
    grid=(NP // 1024,),
    in_specs=[_rowblock(1024, 128)] * 4
    + [_rowblock(1024, 128),
       _full((H, H)), _full((1, H)), _full((H, H)), _full((1, H))],
    out_specs=[_rowblock(1024, H)] * 2,
    out_shape=(jax.ShapeDtypeStruct((NP, H), F32),
               jax.ShapeDtypeStruct((NP, H), F32)),
)

_tc_mlp = pl.pallas_call(
    _tc_mlp_body,
    grid=(16,),
    in_specs=[_rowblock(1024, H), _rowblock(1024, H)]
    + [_full((H, H)), _full((1, H))] * 4,
    out_specs=[_rowblock(1024, H)] * 2,
    out_shape=(jax.ShapeDtypeStruct((16384, H), F32),
               jax.ShapeDtypeStruct((16384, H), F32)),
)


def kernel(x_F, x_S, edge_index, sub_edges,
           W1_F, b1_F, W2_F, b2_F, W1_S, b1_S, W2_S, b2_S,
           Wd1_FS, bd1_FS, Wd2_FS, bd2_FS, Wd1_SF, bd1_SF, Wd2_SF, bd2_SF):
  src, dst = edge_index[0], edge_index[1]
  pad = EROWS * 128 - E
  fill = jnp.full((pad,), N, jnp.int32)
  src_p = jnp.concatenate([src, fill]).reshape(EROWS, 128)
  dst_p = jnp.concatenate([dst, fill]).reshape(EROWS, 128)
  q2d = sub_edges[0].reshape(128, 128)

  cnt_out, cnt_in = _sc_degrees(src_p, dst_p)
  h_f, h_s = _tc_prep(x_F, x_S, cnt_out)
  agg_l, agg_r = _sc_agg(h_f, h_s, src_p, dst_p)
  h1fl, h1fr, h1sl, h1sr = _tc_l1(
      agg_l, agg_r, cnt_in, cnt_out,
      W1_F, b1_F.reshape(1, H), W1_S, b1_S.reshape(1, H))
  a2fl, a2fr = _sc_agg(h1fl, h1fr, src_p, dst_p)
  a2sl, a2sr = _sc_agg(h1sl, h1sr, src_p, dst_p)
  z_f, z_s = _tc_l2(
      a2fl, a2fr, a2sl, a2sr, cnt_in,
      W2_F, b2_F.reshape(1, H), W2_S, b2_S.reshape(1, H))
  zq_f, zq_s = _sc_qgather(z_f, z_s, q2d)
  z_sf, z_fs = _tc_mlp(
      zq_f, zq_s,
      Wd1_SF, bd1_SF.reshape(1, H), Wd2_SF, bd2_SF.reshape(1, H),
      Wd1_FS, bd1_FS.reshape(1, H), Wd2_FS, bd2_FS.reshape(1, H))
  return (zq_f, zq_s, z_sf, z_fs)


# speedup vs baseline: 1.1307x; 1.1307x over previous
import functools

import jax
import jax.numpy as jnp
from jax import lax
from jax.experimental import pallas as pl
from jax.experimental.pallas import tpu as pltpu
from jax.experimental.pallas import tpu_sc as plsc

N = 10000
NP = 10240
E = 320000
D = 128
H = 256
NC, NS, L = 2, 16, 16
RPW = 80
EROWS = NC * NS * RPW
RPS = EROWS // NS
_IDXC = 40
RPT = NP // NS

F32 = jnp.float32


@functools.lru_cache(maxsize=None)
def _sc_mesh():
  return plsc.VectorSubcoreMesh(
      core_axis_name="c", subcore_axis_name="s", num_cores=NC, num_subcores=NS)


def _zero_block(buf, nrows, ncols):
  def fill(i, _):
    for c in range(ncols // L):
      buf[i, c * L:(c + 1) * L] = jnp.zeros((L,), F32)
    return 0
  lax.fori_loop(0, nrows, fill, 0)


def _sc_degrees_body(src_hbm, dst_hbm, cnt_out_hbm, cnt_in_hbm,
                     idx_v, ones_v, zeros_v, accum):
  cid = lax.axis_index("c")
  sid = lax.axis_index("s")

  _zero_block(zeros_v, 128, 128)

  def fill(i, _):
    for c in range(128 // L):
      ones_v[i, c * L:(c + 1) * L] = jnp.ones((L,), F32)
    return 0
  lax.fori_loop(0, 128, fill, 0)

  base = sid * RPT
  for k in range(RPT // 128):
    pltpu.sync_copy(zeros_v, accum.at[pl.ds(base + k * 128, 128)])
  plsc.subcore_barrier()

  def run(e_hbm):
    def outer(oc, _):
      ebase = sid * RPS + oc * _IDXC
      pltpu.sync_copy(e_hbm.at[pl.ds(ebase, _IDXC)], idx_v)

      def step(j, _):
        pltpu.sync_copy(ones_v, accum.at[idx_v.at[j]], add=True)
        return 0
      lax.fori_loop(0, _IDXC, step, 0)
      return 0
    lax.fori_loop(0, RPS // _IDXC, outer, 0)

  @pl.when(cid == 0)
  def _():
    run(src_hbm)

  @pl.when(cid == 1)
  def _():
    run(dst_hbm)

  plsc.subcore_barrier()

  @pl.when(cid == 0)
  def _():
    pltpu.sync_copy(accum.at[pl.ds(base, RPT)], cnt_out_hbm.at[pl.ds(base, RPT)])

  @pl.when(cid == 1)
  def _():
    pltpu.sync_copy(accum.at[pl.ds(base, RPT)], cnt_in_hbm.at[pl.ds(base, RPT)])


def _sc_agg_body(h_l, h_r, src_hbm, dst_hbm, out_l, out_r,
                 idx_s, idx_d, rows_a, rows_b, gsem_a, gsem_b,
                 ssem_a, ssem_b, accum):
  cid = lax.axis_index("c")
  sid = lax.axis_index("s")

  _zero_block(rows_a, 128, 128)
  base = sid * RPT
  for k in range(RPT // 128):
    pltpu.sync_copy(rows_a, accum.at[pl.ds(base + k * 128, 128)])
  plsc.subcore_barrier()

  npairs = _IDXC // 2

  def run(h_ref):
    def outer(oc, _):
      ebase = sid * RPS + oc * _IDXC
      pltpu.sync_copy(src_hbm.at[pl.ds(ebase, _IDXC)], idx_s)
      pltpu.sync_copy(dst_hbm.at[pl.ds(ebase, _IDXC)], idx_d)

      def pair(p, _):
        j0 = 2 * p

        @pl.when(p == 0)
        def _():
          pltpu.async_copy(h_ref.at[idx_s.at[j0]], rows_a, gsem_a)

        pltpu.async_copy(h_ref.at[idx_s.at[j0 + 1]], rows_b, gsem_b)
        pltpu.make_async_copy(h_ref.at[idx_s.at[j0]], rows_a, gsem_a).wait()
        pltpu.sync_copy(rows_a, accum.at[idx_d.at[j0]], add=True)

        @pl.when(p < npairs - 1)
        def _():
          pltpu.async_copy(h_ref.at[idx_s.at[j0 + 2]], rows_a, gsem_a)

        pltpu.make_async_copy(h_ref.at[idx_s.at[j0 + 1]], rows_b, gsem_b).wait()
        pltpu.sync_copy(rows_b, accum.at[idx_d.at[j0 + 1]], add=True)
        return 0
      lax.fori_loop(0, npairs, pair, 0)
      return 0
    lax.fori_loop(0, RPS // _IDXC, outer, 0)

  @pl.when(cid == 0)
  def _():
    run(h_l)

  @pl.when(cid == 1)
  def _():
    run(h_r)

  plsc.subcore_barrier()

  @pl.when(cid == 0)
  def _():
    pltpu.sync_copy(accum.at[pl.ds(base, RPT)], out_l.at[pl.ds(base, RPT)])

  @pl.when(cid == 1)
  def _():
    pltpu.sync_copy(accum.at[pl.ds(base, RPT)], out_r.at[pl.ds(base, RPT)])


def _sc_qgather_body(z_f, z_s, q_hbm, out_f, out_s, idx_v, rows_v, sem):
  cid = lax.axis_index("c")
  sid = lax.axis_index("s")
  w = sid * NC + cid
  pltpu.sync_copy(q_hbm.at[pl.ds((w // 2) * 8, 8)], idx_v)
  half = (w % 2) * 4

  def run(src_ref, dst_ref):
    def step(j, _):
      pltpu.async_copy(src_ref.at[idx_v.at[half + j]], rows_v, sem).wait()
      pltpu.sync_copy(rows_v, dst_ref.at[pl.ds(w * 512 + j * 128, 128)])
      return 0
    lax.fori_loop(0, 4, step, 0)

  run(z_f, out_f)
  run(z_s, out_s)


@functools.lru_cache(maxsize=None)
def _sc_built():
  mesh = _sc_mesh()
  degrees = pl.kernel(
      _sc_degrees_body,
      out_type=(jax.ShapeDtypeStruct((NP, 128), F32),
                jax.ShapeDtypeStruct((NP, 128), F32)),
      mesh=mesh,
      scratch_types=[
          pltpu.VMEM((_IDXC, 128), jnp.int32),
          pltpu.VMEM((128, 128), F32),
          pltpu.VMEM((128, 128), F32),
          pltpu.VMEM_SHARED((NP, 128), F32),
      ],
  )
  agg = pl.kernel(
      _sc_agg_body,
      out_type=(jax.ShapeDtypeStruct((NP, 128), F32),
                jax.ShapeDtypeStruct((NP, 128), F32)),
      mesh=mesh,
      scratch_types=[
          pltpu.VMEM((_IDXC, 128), jnp.int32),
          pltpu.VMEM((_IDXC, 128), jnp.int32),
          pltpu.VMEM((128, 128), F32),
          pltpu.VMEM((128, 128), F32),
          pltpu.SemaphoreType.DMA,
          pltpu.SemaphoreType.DMA,
          pltpu.SemaphoreType.DMA,
          pltpu.SemaphoreType.DMA,
          pltpu.VMEM_SHARED((NP, 128), F32),
      ],
  )
  qgather = pl.kernel(
      _sc_qgather_body,
      out_type=(jax.ShapeDtypeStruct((16384, H), F32),
                jax.ShapeDtypeStruct((16384, H), F32)),
      mesh=mesh,
      scratch_types=[
          pltpu.VMEM((8, 128), jnp.int32),
          pltpu.VMEM((128, H), F32),
          pltpu.SemaphoreType.DMA,
      ],
  )
  return degrees, agg, qgather


def _sc_degrees(*args):
  return _sc_built()[0](*args)


def _sc_agg(*args):
  return _sc_built()[1](*args)


def _sc_qgather(*args):
  return _sc_built()[2](*args)


def _deg_scale(c):
  return lax.rsqrt(jnp.maximum(c[:, 0:1], 1.0))


def _row_mask(i, rows):
  rid = i * rows + lax.broadcasted_iota(jnp.int32, (rows, 1), 0)
  return rid < N


def _elu(x):
  return jnp.where(x > 0, x, jnp.exp(x) - 1.0)


def _tc_prep_body(xf, xs, co, hf, hs):
  i = pl.program_id(0)
  so = _deg_scale(co[...])
  m = _row_mask(i, 1024)
  hf[...] = jnp.where(m, xf[...] * so, 0.0)
  hs[...] = jnp.where(m, xs[...] * so, 0.0)


def _tc_l1_body(al, ar, ci, co, w1f, b1f, w1s, b1s,
                ofl, ofr, osl, osr):
  i = pl.program_id(0)
  di = _deg_scale(ci[...])
  do = _deg_scale(co[...])
  m = _row_mask(i, 1024)
  hf = _elu(jnp.dot(al[...] * di, w1f[...],
                    preferred_element_type=F32) + b1f[...])
  hf = jnp.where(m, hf * do, 0.0)
  ofl[...] = hf[:, :128]
  ofr[...] = hf[:, 128:]
  hs = _elu(jnp.dot(ar[...] * di, w1s[...],
                    preferred_element_type=F32) + b1s[...])
  hs = jnp.where(m, hs * do, 0.0)
  osl[...] = hs[:, :128]
  osr[...] = hs[:, 128:]


def _tc_l2_body(fl, fr, sl, sr, ci, w2f, b2f, w2s, b2s, zf, zs):
  di = _deg_scale(ci[...])
  wf = w2f[...]
  ws = w2s[...]
  zf[...] = (jnp.dot(fl[...] * di, wf[:128, :], preferred_element_type=F32)
             + jnp.dot(fr[...] * di, wf[128:, :], preferred_element_type=F32)
             + b2f[...])
  zs[...] = (jnp.dot(sl[...] * di, ws[:128, :], preferred_element_type=F32)
             + jnp.dot(sr[...] * di, ws[128:, :], preferred_element_type=F32)
             + b2s[...])


def _tc_mlp_body(zqf, zqs, wd1_sf, bd1_sf, wd2_sf, bd2_sf,
                 wd1_fs, bd1_fs, wd2_fs, bd2_fs, zsf, zfs):
  h = _elu(jnp.dot(zqs[...], wd1_sf[...], preferred_element_type=F32)
           + bd1_sf[...])
  zsf[...] = _elu(jnp.dot(h, wd2_sf[...], preferred_element_type=F32)
                  + bd2_sf[...])
  h = _elu(jnp.dot(zqf[...], wd1_fs[...], preferred_element_type=F32)
           + bd1_fs[...])
  zfs[...] = _elu(jnp.dot(h, wd2_fs[...], preferred_element_type=F32)
                  + bd2_fs[...])


def _rowblock(rows, cols):
  return pl.BlockSpec((rows, cols), lambda i: (i, 0))


def _full(shape):
  return pl.BlockSpec(shape, lambda i: tuple(0 for _ in shape))


_tc_prep = pl.pallas_call(
    _tc_prep_body,
    grid=(NP // 1024,),
    in_specs=[_rowblock(1024, D), _rowblock(1024, D), _rowblock(1024, 128)],
    out_specs=[_rowblock(1024, D), _rowblock(1024, D)],
    out_shape=(jax.ShapeDtypeStruct((NP, D), F32),
               jax.ShapeDtypeStruct((NP, D), F32)),
)

_tc_l1 = pl.pallas_call(
    _tc_l1_body,
    grid=(NP // 1024,),
    in_specs=[_rowblock(1024, 128), _rowblock(1024, 128),
              _rowblock(1024, 128), _rowblock(1024, 128),
              _full((D, H)), _full((1, H)), _full((D, H)), _full((1, H))],
    out_specs=[_rowblock(1024, 128)] * 4,
    out_shape=tuple(jax.ShapeDtypeStruct((NP, 128), F32) for _ in range(4)),
)

_tc_l2 = pl.pallas_call(
    _tc_l2_body,
    grid=(NP // 1024,),
    in_specs=[_rowblock(1024, 128)] * 4
    + [_rowblock(1024, 128),
       _full((H, H)), _full((1, H)), _full((H, H)), _full((1, H))],
    out_specs=[_rowblock(1024, H)] * 2,
    out_shape=(jax.ShapeDtypeStruct((NP, H), F32),
               jax.ShapeDtypeStruct((NP, H), F32)),
)

_tc_mlp = pl.pallas_call(
    _tc_mlp_body,
    grid=(16,),
    in_specs=[_rowblock(1024, H), _rowblock(1024, H)]
    + [_full((H, H)), _full((1, H))] * 4,
    out_specs=[_rowblock(1024, H)] * 2,
    out_shape=(jax.ShapeDtypeStruct((16384, H), F32),
               jax.ShapeDtypeStruct((16384, H), F32)),
)


def kernel(x_F, x_S, edge_index, sub_edges,
           W1_F, b1_F, W2_F, b2_F, W1_S, b1_S, W2_S, b2_S,
           Wd1_FS, bd1_FS, Wd2_FS, bd2_FS, Wd1_SF, bd1_SF, Wd2_SF, bd2_SF):
  src, dst = edge_index[0], edge_index[1]
  pad = EROWS * 128 - E
  fill = jnp.full((pad,), N, jnp.int32)
  src_p = jnp.concatenate([src, fill]).reshape(EROWS, 128)
  dst_p = jnp.concatenate([dst, fill]).reshape(EROWS, 128)
  q2d = sub_edges[0].reshape(128, 128)

  cnt_out, cnt_in = _sc_degrees(src_p, dst_p)
  h_f, h_s = _tc_prep(x_F, x_S, cnt_out)
  agg_l, agg_r = _sc_agg(h_f, h_s, src_p, dst_p)
  h1fl, h1fr, h1sl, h1sr = _tc_l1(
      agg_l, agg_r, cnt_in, cnt_out,
      W1_F, b1_F.reshape(1, H), W1_S, b1_S.reshape(1, H))
  a2fl, a2fr = _sc_agg(h1fl, h1fr, src_p, dst_p)
  a2sl, a2sr = _sc_agg(h1sl, h1sr, src_p, dst_p)
  z_f, z_s = _tc_l2(
      a2fl, a2fr, a2sl, a2sr, cnt_in,
      W2_F, b2_F.reshape(1, H), W2_S, b2_S.reshape(1, H))
  zq_f, zq_s = _sc_qgather(z_f, z_s, q2d)
  z_sf, z_fs = _tc_mlp(
      zq_f, zq_s,
      Wd1_SF, bd1_SF.reshape(1, H), Wd2_SF, bd2_SF.reshape(1, H),
      Wd1_FS, bd1_FS.reshape(1, H), Wd2_FS, bd2_FS.reshape(1, H))
  return (zq_f, zq_s, z_sf, z_fs)


# speedup vs baseline: 1.1437x; 1.0115x over previous
import functools

import jax
import jax.numpy as jnp
from jax import lax
from jax.experimental import pallas as pl
from jax.experimental.pallas import tpu as pltpu
from jax.experimental.pallas import tpu_sc as plsc

N = 10000
NP = 10240
E = 320000
D = 128
H = 256
NC, NS, L = 2, 16, 16
RPW = 80
EROWS = NC * NS * RPW
RPS = EROWS // NS
_IDXC = 40
RPT = NP // NS

F32 = jnp.float32


@functools.lru_cache(maxsize=None)
def _sc_mesh():
  return plsc.VectorSubcoreMesh(
      core_axis_name="c", subcore_axis_name="s", num_cores=NC, num_subcores=NS)


def _zero_block(buf, nrows, ncols):
  def fill(i, _):
    for c in range(ncols // L):
      buf[i, c * L:(c + 1) * L] = jnp.zeros((L,), F32)
    return 0
  lax.fori_loop(0, nrows, fill, 0)


def _sc_degrees_body(src_hbm, dst_hbm, cnt_out_hbm, cnt_in_hbm,
                     idx_v, ones_v, zeros_v, accum):
  cid = lax.axis_index("c")
  sid = lax.axis_index("s")

  _zero_block(zeros_v, 128, 128)

  def fill(i, _):
    for c in range(128 // L):
      ones_v[i, c * L:(c + 1) * L] = jnp.ones((L,), F32)
    return 0
  lax.fori_loop(0, 128, fill, 0)

  base = sid * RPT
  for k in range(RPT // 128):
    pltpu.sync_copy(zeros_v, accum.at[pl.ds(base + k * 128, 128)])
  plsc.subcore_barrier()

  def run(e_hbm):
    def outer(oc, _):
      ebase = sid * RPS + oc * _IDXC
      pltpu.sync_copy(e_hbm.at[pl.ds(ebase, _IDXC)], idx_v)

      def step(j, _):
        pltpu.sync_copy(ones_v, accum.at[idx_v.at[j]], add=True)
        return 0
      lax.fori_loop(0, _IDXC, step, 0)
      return 0
    lax.fori_loop(0, RPS // _IDXC, outer, 0)

  @pl.when(cid == 0)
  def _():
    run(src_hbm)

  @pl.when(cid == 1)
  def _():
    run(dst_hbm)

  plsc.subcore_barrier()

  @pl.when(cid == 0)
  def _():
    pltpu.sync_copy(accum.at[pl.ds(base, RPT)], cnt_out_hbm.at[pl.ds(base, RPT)])

  @pl.when(cid == 1)
  def _():
    pltpu.sync_copy(accum.at[pl.ds(base, RPT)], cnt_in_hbm.at[pl.ds(base, RPT)])


def _agg_phase(cid, sid, h_l, h_r, src_hbm, dst_hbm, out_l, out_r,
               idx_s, idx_d, rows_a, rows_b, gsem_a, gsem_b, accum):
  _zero_block(rows_a, 128, 128)
  base = sid * RPT
  for k in range(RPT // 128):
    pltpu.sync_copy(rows_a, accum.at[pl.ds(base + k * 128, 128)])
  plsc.subcore_barrier()

  npairs = _IDXC // 2

  def run(h_ref):
    def outer(oc, _):
      ebase = sid * RPS + oc * _IDXC
      pltpu.sync_copy(src_hbm.at[pl.ds(ebase, _IDXC)], idx_s)
      pltpu.sync_copy(dst_hbm.at[pl.ds(ebase, _IDXC)], idx_d)

      def pair(p, _):
        j0 = 2 * p

        @pl.when(p == 0)
        def _():
          pltpu.async_copy(h_ref.at[idx_s.at[j0]], rows_a, gsem_a)

        pltpu.async_copy(h_ref.at[idx_s.at[j0 + 1]], rows_b, gsem_b)
        pltpu.make_async_copy(h_ref.at[idx_s.at[j0]], rows_a, gsem_a).wait()
        pltpu.sync_copy(rows_a, accum.at[idx_d.at[j0]], add=True)

        @pl.when(p < npairs - 1)
        def _():
          pltpu.async_copy(h_ref.at[idx_s.at[j0 + 2]], rows_a, gsem_a)

        pltpu.make_async_copy(h_ref.at[idx_s.at[j0 + 1]], rows_b, gsem_b).wait()
        pltpu.sync_copy(rows_b, accum.at[idx_d.at[j0 + 1]], add=True)
        return 0
      lax.fori_loop(0, npairs, pair, 0)
      return 0
    lax.fori_loop(0, RPS // _IDXC, outer, 0)

  @pl.when(cid == 0)
  def _():
    run(h_l)

  @pl.when(cid == 1)
  def _():
    run(h_r)

  plsc.subcore_barrier()

  @pl.when(cid == 0)
  def _():
    pltpu.sync_copy(accum.at[pl.ds(base, RPT)], out_l.at[pl.ds(base, RPT)])

  @pl.when(cid == 1)
  def _():
    pltpu.sync_copy(accum.at[pl.ds(base, RPT)], out_r.at[pl.ds(base, RPT)])


def _sc_agg_body(h_l, h_r, src_hbm, dst_hbm, out_l, out_r,
                 idx_s, idx_d, rows_a, rows_b, gsem_a, gsem_b, accum):
  cid = lax.axis_index("c")
  sid = lax.axis_index("s")
  _agg_phase(cid, sid, h_l, h_r, src_hbm, dst_hbm, out_l, out_r,
             idx_s, idx_d, rows_a, rows_b, gsem_a, gsem_b, accum)


def _sc_agg2_body(hf_l, hf_r, hs_l, hs_r, src_hbm, dst_hbm,
                  of_l, of_r, os_l, os_r,
                  idx_s, idx_d, rows_a, rows_b, gsem_a, gsem_b, accum):
  cid = lax.axis_index("c")
  sid = lax.axis_index("s")
  _agg_phase(cid, sid, hf_l, hf_r, src_hbm, dst_hbm, of_l, of_r,
             idx_s, idx_d, rows_a, rows_b, gsem_a, gsem_b, accum)
  plsc.subcore_barrier()
  _agg_phase(cid, sid, hs_l, hs_r, src_hbm, dst_hbm, os_l, os_r,
             idx_s, idx_d, rows_a, rows_b, gsem_a, gsem_b, accum)


def _sc_qgather_body(z_f, z_s, q_hbm, out_f, out_s, idx_v, rows_v, sem):
  cid = lax.axis_index("c")
  sid = lax.axis_index("s")
  w = sid * NC + cid
  pltpu.sync_copy(q_hbm.at[pl.ds((w // 2) * 8, 8)], idx_v)
  half = (w % 2) * 4

  def run(src_ref, dst_ref):
    def step(j, _):
      pltpu.async_copy(src_ref.at[idx_v.at[half + j]], rows_v, sem).wait()
      pltpu.sync_copy(rows_v, dst_ref.at[pl.ds(w * 512 + j * 128, 128)])
      return 0
    lax.fori_loop(0, 4, step, 0)

  run(z_f, out_f)
  run(z_s, out_s)


@functools.lru_cache(maxsize=None)
def _sc_built():
  mesh = _sc_mesh()
  degrees = pl.kernel(
      _sc_degrees_body,
      out_type=(jax.ShapeDtypeStruct((NP, 128), F32),
                jax.ShapeDtypeStruct((NP, 128), F32)),
      mesh=mesh,
      scratch_types=[
          pltpu.VMEM((_IDXC, 128), jnp.int32),
          pltpu.VMEM((128, 128), F32),
          pltpu.VMEM((128, 128), F32),
          pltpu.VMEM_SHARED((NP, 128), F32),
      ],
  )
  _agg_scratch = [
      pltpu.VMEM((_IDXC, 128), jnp.int32),
      pltpu.VMEM((_IDXC, 128), jnp.int32),
      pltpu.VMEM((128, 128), F32),
      pltpu.VMEM((128, 128), F32),
      pltpu.SemaphoreType.DMA,
      pltpu.SemaphoreType.DMA,
      pltpu.VMEM_SHARED((NP, 128), F32),
  ]
  agg = pl.kernel(
      _sc_agg_body,
      out_type=(jax.ShapeDtypeStruct((NP, 128), F32),
                jax.ShapeDtypeStruct((NP, 128), F32)),
      mesh=mesh,
      scratch_types=_agg_scratch,
  )
  agg2 = pl.kernel(
      _sc_agg2_body,
      out_type=tuple(jax.ShapeDtypeStruct((NP, 128), F32) for _ in range(4)),
      mesh=mesh,
      scratch_types=_agg_scratch,
  )
  qgather = pl.kernel(
      _sc_qgather_body,
      out_type=(jax.ShapeDtypeStruct((16384, H), F32),
                jax.ShapeDtypeStruct((16384, H), F32)),
      mesh=mesh,
      scratch_types=[
          pltpu.VMEM((8, 128), jnp.int32),
          pltpu.VMEM((128, H), F32),
          pltpu.SemaphoreType.DMA,
      ],
  )
  return degrees, agg, agg2, qgather


def _sc_degrees(*args):
  return _sc_built()[0](*args)


def _sc_agg(*args):
  return _sc_built()[1](*args)


def _sc_agg2(*args):
  return _sc_built()[2](*args)


def _sc_qgather(*args):
  return _sc_built()[3](*args)


def _deg_scale(c):
  return lax.rsqrt(jnp.maximum(c[:, 0:1], 1.0))


def _row_mask(i, rows):
  rid = i * rows + lax.broadcasted_iota(jnp.int32, (rows, 1), 0)
  return rid < N


def _elu(x):
  return jnp.where(x > 0, x, jnp.exp(x) - 1.0)


def _tc_prep_body(xf, xs, co, hf, hs):
  i = pl.program_id(0)
  so = _deg_scale(co[...])
  m = _row_mask(i, 1024)
  hf[...] = jnp.where(m, xf[...] * so, 0.0)
  hs[...] = jnp.where(m, xs[...] * so, 0.0)


def _tc_l1_body(al, ar, ci, co, w1f, b1f, w1s, b1s,
                ofl, ofr, osl, osr):
  i = pl.program_id(0)
  di = _deg_scale(ci[...])
  do = _deg_scale(co[...])
  m = _row_mask(i, 1024)
  hf = _elu(jnp.dot(al[...] * di, w1f[...],
                    preferred_element_type=F32) + b1f[...])
  hf = jnp.where(m, hf * do, 0.0)
  ofl[...] = hf[:, :128]
  ofr[...] = hf[:, 128:]
  hs = _elu(jnp.dot(ar[...] * di, w1s[...],
                    preferred_element_type=F32) + b1s[...])
  hs = jnp.where(m, hs * do, 0.0)
  osl[...] = hs[:, :128]
  osr[...] = hs[:, 128:]


def _tc_l2_body(fl, fr, sl, sr, ci, w2f, b2f, w2s, b2s, zf, zs):
  di = _deg_scale(ci[...])
  wf = w2f[...]
  ws = w2s[...]
  zf[...] = (jnp.dot(fl[...] * di, wf[:128, :], preferred_element_type=F32)
             + jnp.dot(fr[...] * di, wf[128:, :], preferred_element_type=F32)
             + b2f[...])
  zs[...] = (jnp.dot(sl[...] * di, ws[:128, :], preferred_element_type=F32)
             + jnp.dot(sr[...] * di, ws[128:, :], preferred_element_type=F32)
             + b2s[...])


def _tc_mlp_body(zqf, zqs, wd1_sf, bd1_sf, wd2_sf, bd2_sf,
                 wd1_fs, bd1_fs, wd2_fs, bd2_fs, zsf, zfs):
  h = _elu(jnp.dot(zqs[...], wd1_sf[...], preferred_element_type=F32)
           + bd1_sf[...])
  zsf[...] = _elu(jnp.dot(h, wd2_sf[...], preferred_element_type=F32)
                  + bd2_sf[...])
  h = _elu(jnp.dot(zqf[...], wd1_fs[...], preferred_element_type=F32)
           + bd1_fs[...])
  zfs[...] = _elu(jnp.dot(h, wd2_fs[...], preferred_element_type=F32)
                  + bd2_fs[...])


def _rowblock(rows, cols):
  return pl.BlockSpec((rows, cols), lambda i: (i, 0))


def _full(shape):
  return pl.BlockSpec(shape, lambda i: tuple(0 for _ in shape))


_tc_prep = pl.pallas_call(
    _tc_prep_body,
    grid=(NP // 1024,),
    in_specs=[_rowblock(1024, D), _rowblock(1024, D), _rowblock(1024, 128)],
    out_specs=[_rowblock(1024, D), _rowblock(1024, D)],
    out_shape=(jax.ShapeDtypeStruct((NP, D), F32),
               jax.ShapeDtypeStruct((NP, D), F32)),
)

_tc_l1 = pl.pallas_call(
    _tc_l1_body,
    grid=(NP // 1024,),
    in_specs=[_rowblock(1024, 128), _rowblock(1024, 128),
              _rowblock(1024, 128), _rowblock(1024, 128),
              _full((D, H)), _full((1, H)), _full((D, H)), _full((1, H))],
    out_specs=[_rowblock(1024, 128)] * 4,
    out_shape=tuple(jax.ShapeDtypeStruct((NP, 128), F32) for _ in range(4)),
)

_tc_l2 = pl.pallas_call(
    _tc_l2_body,
    grid=(NP // 1024,),
    in_specs=[_rowblock(1024, 128)] * 4
    + [_rowblock(1024, 128),
       _full((H, H)), _full((1, H)), _full((H, H)), _full((1, H))],
    out_specs=[_rowblock(1024, H)] * 2,
    out_shape=(jax.ShapeDtypeStruct((NP, H), F32),
               jax.ShapeDtypeStruct((NP, H), F32)),
)

_tc_mlp = pl.pallas_call(
    _tc_mlp_body,
    grid=(16,),
    in_specs=[_rowblock(1024, H), _rowblock(1024, H)]
    + [_full((H, H)), _full((1, H))] * 4,
    out_specs=[_rowblock(1024, H)] * 2,
    out_shape=(jax.ShapeDtypeStruct((16384, H), F32),
               jax.ShapeDtypeStruct((16384, H), F32)),
)


def kernel(x_F, x_S, edge_index, sub_edges,
           W1_F, b1_F, W2_F, b2_F, W1_S, b1_S, W2_S, b2_S,
           Wd1_FS, bd1_FS, Wd2_FS, bd2_FS, Wd1_SF, bd1_SF, Wd2_SF, bd2_SF):
  src, dst = edge_index[0], edge_index[1]
  pad = EROWS * 128 - E
  fill = jnp.full((pad,), N, jnp.int32)
  src_p = jnp.concatenate([src, fill]).reshape(EROWS, 128)
  dst_p = jnp.concatenate([dst, fill]).reshape(EROWS, 128)
  q2d = sub_edges[0].reshape(128, 128)

  cnt_out, cnt_in = _sc_degrees(src_p, dst_p)
  h_f, h_s = _tc_prep(x_F, x_S, cnt_out)
  agg_l, agg_r = _sc_agg(h_f, h_s, src_p, dst_p)
  h1fl, h1fr, h1sl, h1sr = _tc_l1(
      agg_l, agg_r, cnt_in, cnt_out,
      W1_F, b1_F.reshape(1, H), W1_S, b1_S.reshape(1, H))
  a2fl, a2fr, a2sl, a2sr = _sc_agg2(h1fl, h1fr, h1sl, h1sr, src_p, dst_p)
  z_f, z_s = _tc_l2(
      a2fl, a2fr, a2sl, a2sr, cnt_in,
      W2_F, b2_F.reshape(1, H), W2_S, b2_S.reshape(1, H))
  zq_f, zq_s = _sc_qgather(z_f, z_s, q2d)
  z_sf, z_fs = _tc_mlp(
      zq_f, zq_s,
      Wd1_SF, bd1_SF.reshape(1, H), Wd2_SF, bd2_SF.reshape(1, H),
      Wd1_FS, bd1_FS.reshape(1, H), Wd2_FS, bd2_FS.reshape(1, H))
  return (zq_f, zq_s, z_sf, z_fs)
